# baseline (speedup 1.0000x reference)
import functools

import jax
import jax.numpy as jnp
from jax import lax
from jax.experimental import pallas as pl
from jax.experimental.pallas import tpu as pltpu
from jax.experimental.pallas import tpu_sc as plsc

N = 10000
NP = 10240
E = 320000
D = 128
G = 64

BLK = 1024
GRID = NP // BLK

NW = 32
EPW = E // NW
K = 128
EP = NW * 10240
EPWP = EP // NW
NCH = EPWP // K
NBUF = 2
NGRP = NCH // NBUF
DEGG = 5
RPS = NP // 16


def _prep_body(parts_ref, x_ref, w_ref, dinv_ref, y_ref):
    deg = parts_ref[0, :, 0:1] + parts_ref[1, :, 0:1] + 1.0
    dinv_col = lax.rsqrt(deg)
    dinv_ref[...] = dinv_col
    y = jax.lax.dot_general(x_ref[...], w_ref[...], (((1,), (0,)), ((), ())),
                            preferred_element_type=jnp.float32)
    y_ref[...] = y * dinv_col


def _prep(deg_parts, x, w0):
    return pl.pallas_call(
        _prep_body,
        grid=(GRID,),
        in_specs=[
            pl.BlockSpec((2, BLK, D), lambda i: (0, i, 0)),
            pl.BlockSpec((BLK, D), lambda i: (i, 0)),
            pl.BlockSpec((D, D), lambda i: (0, 0)),
        ],
        out_specs=[
            pl.BlockSpec((BLK, 1), lambda i: (i, 0)),
            pl.BlockSpec((BLK, D), lambda i: (i, 0)),
        ],
        out_shape=[
            jax.ShapeDtypeStruct((NP, 1), jnp.float32),
            jax.ShapeDtypeStruct((NP, D), jnp.float32),
        ],
    )(deg_parts, x, w0)


def _layer_body(p_ref, y_ref, dinv_ref, b_ref, w_ref, out_ref):
    agg = p_ref[0] + p_ref[1] + y_ref[...]
    h = jnp.maximum(agg * dinv_ref[...] + b_ref[...], 0.0)
    out = jax.lax.dot_general(h, w_ref[...], (((1,), (0,)), ((), ())),
                              preferred_element_type=jnp.float32)
    out_ref[...] = out * dinv_ref[...]


def _layer(p, y, dinv, b, w):
    return pl.pallas_call(
        _layer_body,
        grid=(GRID,),
        in_specs=[
            pl.BlockSpec((2, BLK, D), lambda i: (0, i, 0)),
            pl.BlockSpec((BLK, D), lambda i: (i, 0)),
            pl.BlockSpec((BLK, 1), lambda i: (i, 0)),
            pl.BlockSpec((1, D), lambda i: (0, 0)),
            pl.BlockSpec((D, D), lambda i: (0, 0)),
        ],
        out_specs=pl.BlockSpec((BLK, D), lambda i: (i, 0)),
        out_shape=jax.ShapeDtypeStruct((NP, D), jnp.float32),
    )(p, y, dinv, b, w)


def _readout_body(p_ref, y_ref, dinv_ref, b_ref, batch_ref, w1_ref, b1_ref,
                  w2_ref, b2_ref, out_ref, gmax, gsum, cnt):
    i = pl.program_id(0)

    @pl.when(i == 0)
    def _init():
        gmax[...] = jnp.full((G, D), -jnp.inf, jnp.float32)
        gsum[...] = jnp.zeros((G, D), jnp.float32)
        cnt[...] = jnp.zeros((G, 1), jnp.float32)

    agg = p_ref[0] + p_ref[1] + y_ref[...]
    h = jnp.maximum(agg * dinv_ref[...] + b_ref[...], 0.0)

    bidx = batch_ref[...]
    giota = lax.broadcasted_iota(jnp.int32, (BLK, G), 1)
    onehot = (bidx == giota).astype(jnp.float32)
    gsum[...] += jax.lax.dot_general(onehot, h, (((0,), (0,)), ((), ())),
                                     preferred_element_type=jnp.float32)
    cnt[...] += jax.lax.dot_general(
        onehot, jnp.ones((BLK, 1), jnp.float32), (((0,), (0,)), ((), ())),
        preferred_element_type=jnp.float32)

    def upd(g, _):
        row = jnp.max(jnp.where(bidx == g, h, -jnp.inf), axis=0, keepdims=True)
        gmax[pl.ds(g, 1), :] = jnp.maximum(gmax[pl.ds(g, 1), :], row)
        return 0
    lax.fori_loop(0, G, upd, 0)

    @pl.when(i == GRID - 1)
    def _fin():
        gmean = gsum[...] / jnp.maximum(cnt[...], 1.0)
        hcat = jnp.concatenate([gmax[...], gmean], axis=1)
        h1 = jax.lax.dot_general(hcat, w1_ref[...], (((1,), (0,)), ((), ())),
                                 preferred_element_type=jnp.float32)
        h1 = jnp.maximum(h1 + b1_ref[...], 0.0)
        out = jax.lax.dot_general(h1, w2_ref[...], (((1,), (0,)), ((), ())),
                                  preferred_element_type=jnp.float32)
        out_ref[...] = out + b2_ref[...]


def _readout(p, y, dinv, b, batch_col, w1, b1, w2, b2):
    return pl.pallas_call(
        _readout_body,
        grid=(GRID,),
        in_specs=[
            pl.BlockSpec((2, BLK, D), lambda i: (0, i, 0)),
            pl.BlockSpec((BLK, D), lambda i: (i, 0)),
            pl.BlockSpec((BLK, 1), lambda i: (i, 0)),
            pl.BlockSpec((1, D), lambda i: (0, 0)),
            pl.BlockSpec((BLK, 1), lambda i: (i, 0)),
            pl.BlockSpec((2 * D, D), lambda i: (0, 0)),
            pl.BlockSpec((1, D), lambda i: (0, 0)),
            pl.BlockSpec((D, 1), lambda i: (0, 0)),
            pl.BlockSpec((1, 1), lambda i: (0, 0)),
        ],
        out_specs=pl.BlockSpec((G, 1), lambda i: (0, 0)),
        out_shape=jax.ShapeDtypeStruct((G, 1), jnp.float32),
        scratch_shapes=[
            pltpu.VMEM((G, D), jnp.float32),
            pltpu.VMEM((G, D), jnp.float32),
            pltpu.VMEM((G, 1), jnp.float32),
        ],
    )(p, y, dinv, b, batch_col, w1, b1, w2, b2)


_sc_mesh = plsc.VectorSubcoreMesh(core_axis_name="c", subcore_axis_name="s")


def _sc_deg(dsts, ones, zeros):
    @functools.partial(
        pl.kernel, mesh=_sc_mesh,
        out_type=jax.ShapeDtypeStruct((2, NP, D), jnp.float32),
        scratch_types=[
            pltpu.VMEM((NCH, K), jnp.int32),
            pltpu.VMEM((K, D), jnp.float32),
            pltpu.VMEM_SHARED((NP, D), jnp.float32),
            pltpu.SemaphoreType.DMA,
        ],
    )
    def deg_kernel(dst_hbm, ones_hbm, z_hbm, out_hbm, dst_v, ones_v, acc, sem):
        c = lax.axis_index("c")
        s = lax.axis_index("s")
        w = s * 2 + c
        pltpu.sync_copy(dst_hbm.at[w], dst_v)
        pltpu.sync_copy(ones_hbm, ones_v)
        pltpu.sync_copy(z_hbm.at[pl.ds(s * RPS, RPS)], acc.at[pl.ds(s * RPS, RPS)])
        plsc.subcore_barrier()

        def body(gi, carry):
            base = gi * DEGG
            descs = [pltpu.async_copy(ones_v, acc.at[dst_v.at[base + b]],
                                      sem, add=True) for b in range(DEGG)]
            for d in descs:
                d.wait()
            return carry
        lax.fori_loop(0, NCH // DEGG, body, 0)
        plsc.subcore_barrier()
        pltpu.sync_copy(acc.at[pl.ds(s * RPS, RPS)],
                        out_hbm.at[c, pl.ds(s * RPS, RPS)])

    return deg_kernel(dsts, ones, zeros)


def _sc_agg(y, srcs, dsts, zeros):
    @functools.partial(
        pl.kernel, mesh=_sc_mesh,
        out_type=jax.ShapeDtypeStruct((2, NP, D), jnp.float32),
        scratch_types=(
            [pltpu.VMEM((NCH, K), jnp.int32)]
            + [pltpu.VMEM((K,), jnp.int32) for _ in range(NBUF)]
            + [pltpu.VMEM((K, D), jnp.float32) for _ in range(NBUF)]
            + [pltpu.VMEM_SHARED((NP, D), jnp.float32)]
            + [pltpu.SemaphoreType.DMA for _ in range(2)]
        ),
    )
    def agg_kernel(y_hbm, src_hbm, dst_hbm, z_hbm, out_hbm, *rest):
        dst_v = rest[0]
        sidx = rest[1:NBUF + 1]
        bufs = rest[NBUF + 1:2 * NBUF + 1]
        acc = rest[2 * NBUF + 1]
        gsem = rest[2 * NBUF + 2]
        ssem = rest[2 * NBUF + 3]
        c = lax.axis_index("c")
        s = lax.axis_index("s")
        w = s * 2 + c
        pltpu.sync_copy(dst_hbm.at[w], dst_v)
        pltpu.sync_copy(z_hbm.at[pl.ds(s * RPS, RPS)], acc.at[pl.ds(s * RPS, RPS)])
        plsc.subcore_barrier()

        for b in range(NBUF):
            pltpu.sync_copy(src_hbm.at[w, b], sidx[b])
            pltpu.async_copy(y_hbm.at[sidx[b]], bufs[b], gsem)

        def body(gi, carry):
            base = gi * NBUF
            descs = []
            for b in range(NBUF):
                j = base + b
                pltpu.make_async_copy(y_hbm.at[sidx[b]], bufs[b],
                                      gsem).wait()
                descs.append(pltpu.async_copy(bufs[b], acc.at[dst_v.at[j]],
                                              ssem, add=True))
            for b in range(NBUF):
                descs[b].wait()
                nj = base + NBUF + b

                @pl.when(nj < NCH)
                def _prefetch():
                    pltpu.sync_copy(src_hbm.at[w, nj], sidx[b])
                    pltpu.async_copy(y_hbm.at[sidx[b]], bufs[b], gsem)
            return carry
        lax.fori_loop(0, NGRP, body, 0)
        plsc.subcore_barrier()
        pltpu.sync_copy(acc.at[pl.ds(s * RPS, RPS)],
                        out_hbm.at[c, pl.ds(s * RPS, RPS)])

    return agg_kernel(y, srcs, dsts, zeros)


def kernel(x, edge_index, batch_index, W0, b0, W1, b1, W2, b2, W3, b3, W4, b4,
           out1_W, out1_b, out2_W, out2_b):
    srcs = jnp.pad(edge_index[0].astype(jnp.int32), (0, EP - E),
                   constant_values=NP - 1).reshape(NW, NCH, K)
    dsts = jnp.pad(edge_index[1].astype(jnp.int32), (0, EP - E),
                   constant_values=NP - 1).reshape(NW, NCH, K)
    xp = jnp.pad(x, ((0, NP - N), (0, 0)))
    batch_p = jnp.pad(batch_index.astype(jnp.int32), (0, NP - N),
                      constant_values=G)
    zeros = jnp.zeros((NP, D), jnp.float32)
    ones = jnp.ones((K, D), jnp.float32)

    deg_parts = _sc_deg(dsts, ones, zeros)
    dinv, y = _prep(deg_parts, xp, W0)

    bs = [b0, b1, b2, b3]
    ws = [W1, W2, W3, W4]
    for k in range(4):
        p = _sc_agg(y, srcs, dsts, zeros)
        y = _layer(p, y, dinv, bs[k].reshape(1, D), ws[k])

    p = _sc_agg(y, srcs, dsts, zeros)
    out = _readout(p, y, dinv, b4.reshape(1, D), batch_p.reshape(NP, 1),
                   out1_W, out1_b.reshape(1, D), out2_W, out2_b.reshape(1, 1))
    return out


# speedup vs baseline: 3.0636x; 3.0636x over previous
import functools

import jax
import jax.numpy as jnp
from jax import lax
from jax.experimental import pallas as pl
from jax.experimental.pallas import tpu as pltpu
from jax.experimental.pallas import tpu_sc as plsc

N = 10000
NP = 10240
E = 320000
D = 128
G = 64

BLK = 1024
GRID = NP // BLK

NW = 32
EPW = E // NW
K = 100
NCH = EPW // K
NBUF = 2
NGRP = NCH // NBUF
DEGG = 5
RPS = NP // 16


def _prep_body(parts_ref, x_ref, w_ref, dinv_ref, y_ref):
    deg = parts_ref[0, :, 0:1] + parts_ref[1, :, 0:1] + 1.0
    dinv_col = lax.rsqrt(deg)
    dinv_ref[...] = dinv_col
    y = jax.lax.dot_general(x_ref[...], w_ref[...], (((1,), (0,)), ((), ())),
                            preferred_element_type=jnp.float32)
    y_ref[...] = y * dinv_col


def _prep(deg_parts, x, w0):
    return pl.pallas_call(
        _prep_body,
        grid=(GRID,),
        in_specs=[
            pl.BlockSpec((2, BLK, D), lambda i: (0, i, 0)),
            pl.BlockSpec((BLK, D), lambda i: (i, 0)),
            pl.BlockSpec((D, D), lambda i: (0, 0)),
        ],
        out_specs=[
            pl.BlockSpec((BLK, 1), lambda i: (i, 0)),
            pl.BlockSpec((BLK, D), lambda i: (i, 0)),
        ],
        out_shape=[
            jax.ShapeDtypeStruct((NP, 1), jnp.float32),
            jax.ShapeDtypeStruct((NP, D), jnp.float32),
        ],
    )(deg_parts, x, w0)


def _layer_body(p_ref, y_ref, dinv_ref, b_ref, w_ref, out_ref):
    agg = p_ref[0] + p_ref[1] + y_ref[...]
    h = jnp.maximum(agg * dinv_ref[...] + b_ref[...], 0.0)
    out = jax.lax.dot_general(h, w_ref[...], (((1,), (0,)), ((), ())),
                              preferred_element_type=jnp.float32)
    out_ref[...] = out * dinv_ref[...]


def _layer(p, y, dinv, b, w):
    return pl.pallas_call(
        _layer_body,
        grid=(GRID,),
        in_specs=[
            pl.BlockSpec((2, BLK, D), lambda i: (0, i, 0)),
            pl.BlockSpec((BLK, D), lambda i: (i, 0)),
            pl.BlockSpec((BLK, 1), lambda i: (i, 0)),
            pl.BlockSpec((1, D), lambda i: (0, 0)),
            pl.BlockSpec((D, D), lambda i: (0, 0)),
        ],
        out_specs=pl.BlockSpec((BLK, D), lambda i: (i, 0)),
        out_shape=jax.ShapeDtypeStruct((NP, D), jnp.float32),
    )(p, y, dinv, b, w)


def _readout_body(p_ref, y_ref, dinv_ref, b_ref, batch_ref, w1_ref, b1_ref,
                  w2_ref, b2_ref, out_ref, gmax, gsum, cnt):
    i = pl.program_id(0)

    @pl.when(i == 0)
    def _init():
        gmax[...] = jnp.full((G, D), -jnp.inf, jnp.float32)
        gsum[...] = jnp.zeros((G, D), jnp.float32)
        cnt[...] = jnp.zeros((G, 1), jnp.float32)

    agg = p_ref[0] + p_ref[1] + y_ref[...]
    h = jnp.maximum(agg * dinv_ref[...] + b_ref[...], 0.0)

    bidx = batch_ref[...]
    giota = lax.broadcasted_iota(jnp.int32, (BLK, G), 1)
    onehot = (bidx == giota).astype(jnp.float32)
    gsum[...] += jax.lax.dot_general(onehot, h, (((0,), (0,)), ((), ())),
                                     preferred_element_type=jnp.float32)
    cnt[...] += jax.lax.dot_general(
        onehot, jnp.ones((BLK, 1), jnp.float32), (((0,), (0,)), ((), ())),
        preferred_element_type=jnp.float32)

    def upd(g, _):
        row = jnp.max(jnp.where(bidx == g, h, -jnp.inf), axis=0, keepdims=True)
        gmax[pl.ds(g, 1), :] = jnp.maximum(gmax[pl.ds(g, 1), :], row)
        return 0
    lax.fori_loop(0, G, upd, 0)

    @pl.when(i == GRID - 1)
    def _fin():
        gmean = gsum[...] / jnp.maximum(cnt[...], 1.0)
        hcat = jnp.concatenate([gmax[...], gmean], axis=1)
        h1 = jax.lax.dot_general(hcat, w1_ref[...], (((1,), (0,)), ((), ())),
                                 preferred_element_type=jnp.float32)
        h1 = jnp.maximum(h1 + b1_ref[...], 0.0)
        out = jax.lax.dot_general(h1, w2_ref[...], (((1,), (0,)), ((), ())),
                                  preferred_element_type=jnp.float32)
        out_ref[...] = out + b2_ref[...]


def _readout(p, y, dinv, b, batch_col, w1, b1, w2, b2):
    return pl.pallas_call(
        _readout_body,
        grid=(GRID,),
        in_specs=[
            pl.BlockSpec((2, BLK, D), lambda i: (0, i, 0)),
            pl.BlockSpec((BLK, D), lambda i: (i, 0)),
            pl.BlockSpec((BLK, 1), lambda i: (i, 0)),
            pl.BlockSpec((1, D), lambda i: (0, 0)),
            pl.BlockSpec((BLK, 1), lambda i: (i, 0)),
            pl.BlockSpec((2 * D, D), lambda i: (0, 0)),
            pl.BlockSpec((1, D), lambda i: (0, 0)),
            pl.BlockSpec((D, 1), lambda i: (0, 0)),
            pl.BlockSpec((1, 1), lambda i: (0, 0)),
        ],
        out_specs=pl.BlockSpec((G, 1), lambda i: (0, 0)),
        out_shape=jax.ShapeDtypeStruct((G, 1), jnp.float32),
        scratch_shapes=[
            pltpu.VMEM((G, D), jnp.float32),
            pltpu.VMEM((G, D), jnp.float32),
            pltpu.VMEM((G, 1), jnp.float32),
        ],
    )(p, y, dinv, b, batch_col, w1, b1, w2, b2)


_sc_mesh = plsc.VectorSubcoreMesh(core_axis_name="c", subcore_axis_name="s")


def _sc_deg(dsts, ones, zeros):
    @functools.partial(
        pl.kernel, mesh=_sc_mesh,
        out_type=jax.ShapeDtypeStruct((2, NP, D), jnp.float32),
        scratch_types=[
            pltpu.VMEM((NCH, K), jnp.int32),
            pltpu.VMEM((K, D), jnp.float32),
            pltpu.VMEM_SHARED((NP, D), jnp.float32),
            pltpu.SemaphoreType.DMA,
        ],
    )
    def deg_kernel(dst_hbm, ones_hbm, z_hbm, out_hbm, dst_v, ones_v, acc, sem):
        c = lax.axis_index("c")
        s = lax.axis_index("s")
        w = s * 2 + c
        pltpu.sync_copy(dst_hbm.at[w], dst_v)
        pltpu.sync_copy(ones_hbm, ones_v)
        pltpu.sync_copy(z_hbm.at[pl.ds(s * RPS, RPS)], acc.at[pl.ds(s * RPS, RPS)])
        plsc.subcore_barrier()

        def body(gi, carry):
            base = gi * DEGG
            descs = [pltpu.async_copy(ones_v, acc.at[dst_v.at[base + b]],
                                      sem, add=True) for b in range(DEGG)]
            for d in descs:
                d.wait()
            return carry
        lax.fori_loop(0, NCH // DEGG, body, 0)
        plsc.subcore_barrier()
        pltpu.sync_copy(acc.at[pl.ds(s * RPS, RPS)],
                        out_hbm.at[c, pl.ds(s * RPS, RPS)])

    return deg_kernel(dsts, ones, zeros)


def _sc_agg(y, srcs, dsts, zeros):
    @functools.partial(
        pl.kernel, mesh=_sc_mesh,
        out_type=jax.ShapeDtypeStruct((2, NP, D), jnp.float32),
        scratch_types=(
            [pltpu.VMEM((NCH, K), jnp.int32)]
            + [pltpu.VMEM((K,), jnp.int32) for _ in range(2 * NBUF)]
            + [pltpu.VMEM((K, D), jnp.float32) for _ in range(NBUF)]
            + [pltpu.VMEM_SHARED((NP, D), jnp.float32)]
            + [pltpu.SemaphoreType.DMA for _ in range(3)]
        ),
    )
    def agg_kernel(y_hbm, src_hbm, dst_hbm, z_hbm, out_hbm, *rest):
        dst_v = rest[0]
        sidxA = rest[1:1 + NBUF]
        sidxB = rest[1 + NBUF:1 + 2 * NBUF]
        bufs = rest[1 + 2 * NBUF:1 + 3 * NBUF]
        acc = rest[1 + 3 * NBUF]
        gsem = rest[2 + 3 * NBUF]
        ssem = rest[3 + 3 * NBUF]
        isem = rest[4 + 3 * NBUF]
        c = lax.axis_index("c")
        s = lax.axis_index("s")
        w = s * 2 + c
        pltpu.sync_copy(dst_hbm.at[w], dst_v)
        pltpu.sync_copy(z_hbm.at[pl.ds(s * RPS, RPS)], acc.at[pl.ds(s * RPS, RPS)])
        plsc.subcore_barrier()

        for b in range(NBUF):
            pltpu.sync_copy(src_hbm.at[w, b], sidxA[b])
            pltpu.async_copy(y_hbm.at[sidxA[b]], bufs[b], gsem)
            pltpu.async_copy(src_hbm.at[w, NBUF + b], sidxB[b], isem)

        def half(base, cur, nxt):
            descs = []
            for b in range(NBUF):
                pltpu.make_async_copy(y_hbm.at[cur[b]], bufs[b], gsem).wait()
                descs.append(pltpu.async_copy(bufs[b], acc.at[dst_v.at[base + b]],
                                              ssem, add=True))
            for b in range(NBUF):
                descs[b].wait()
                gj = base + NBUF + b
                ij = base + 2 * NBUF + b

                @pl.when(gj < NCH)
                def _fire():
                    pltpu.make_async_copy(src_hbm.at[w, gj], nxt[b], isem).wait()
                    pltpu.async_copy(y_hbm.at[nxt[b]], bufs[b], gsem)

                @pl.when(ij < NCH)
                def _prefetch_idx():
                    pltpu.async_copy(src_hbm.at[w, ij], cur[b], isem)

        def body(t, carry):
            half(2 * NBUF * t, sidxA, sidxB)
            half(2 * NBUF * t + NBUF, sidxB, sidxA)
            return carry
        lax.fori_loop(0, NCH // (2 * NBUF), body, 0)
        plsc.subcore_barrier()
        pltpu.sync_copy(acc.at[pl.ds(s * RPS, RPS)],
                        out_hbm.at[c, pl.ds(s * RPS, RPS)])

    return agg_kernel(y, srcs, dsts, zeros)


def kernel(x, edge_index, batch_index, W0, b0, W1, b1, W2, b2, W3, b3, W4, b4,
           out1_W, out1_b, out2_W, out2_b):
    srcs = edge_index[0].astype(jnp.int32).reshape(NW, NCH, K)
    dsts = edge_index[1].astype(jnp.int32).reshape(NW, NCH, K)
    xp = jnp.pad(x, ((0, NP - N), (0, 0)))
    batch_p = jnp.pad(batch_index.astype(jnp.int32), (0, NP - N),
                      constant_values=G)
    zeros = jnp.zeros((NP, D), jnp.float32)
    ones = jnp.ones((K, D), jnp.float32)

    deg_parts = _sc_deg(dsts, ones, zeros)
    dinv, y = _prep(deg_parts, xp, W0)

    bs = [b0, b1, b2, b3]
    ws = [W1, W2, W3, W4]
    for k in range(4):
        p = _sc_agg(y, srcs, dsts, zeros)
        y = _layer(p, y, dinv, bs[k].reshape(1, D), ws[k])

    p = _sc_agg(y, srcs, dsts, zeros)
    out = _readout(p, y, dinv, b4.reshape(1, D), batch_p.reshape(NP, 1),
                   out1_W, out1_b.reshape(1, D), out2_W, out2_b.reshape(1, 1))
    return out


# speedup vs baseline: 3.1493x; 1.0280x over previous
import functools

import jax
import jax.numpy as jnp
from jax import lax
from jax.experimental import pallas as pl
from jax.experimental.pallas import tpu as pltpu
from jax.experimental.pallas import tpu_sc as plsc

N = 10000
NP = 10240
E = 320000
D = 128
G = 64

BLK = 1024
GRID = NP // BLK

NW = 32
EPW = E // NW
K = 100
NCH = EPW // K
NBUF = 2
NGRP = NCH // NBUF
DEGG = 5
RPS = NP // 16
DW = 64


def _prep_body(parts_ref, x_ref, w_ref, dinv_ref, y_ref):
    deg = parts_ref[0, :, 0:1] + parts_ref[1, :, 0:1] + 1.0
    dinv_col = lax.rsqrt(deg)
    dinv_ref[...] = dinv_col
    y = jax.lax.dot_general(x_ref[...], w_ref[...], (((1,), (0,)), ((), ())),
                            preferred_element_type=jnp.float32)
    y_ref[...] = y * dinv_col


def _prep(deg_parts, x, w0):
    return pl.pallas_call(
        _prep_body,
        grid=(GRID,),
        in_specs=[
            pl.BlockSpec((2, BLK, DW), lambda i: (0, i, 0)),
            pl.BlockSpec((BLK, D), lambda i: (i, 0)),
            pl.BlockSpec((D, D), lambda i: (0, 0)),
        ],
        out_specs=[
            pl.BlockSpec((BLK, 1), lambda i: (i, 0)),
            pl.BlockSpec((BLK, D), lambda i: (i, 0)),
        ],
        out_shape=[
            jax.ShapeDtypeStruct((NP, 1), jnp.float32),
            jax.ShapeDtypeStruct((NP, D), jnp.float32),
        ],
    )(deg_parts, x, w0)


def _layer_body(p_ref, y_ref, dinv_ref, b_ref, w_ref, out_ref):
    agg = p_ref[0] + p_ref[1] + y_ref[...]
    h = jnp.maximum(agg * dinv_ref[...] + b_ref[...], 0.0)
    out = jax.lax.dot_general(h, w_ref[...], (((1,), (0,)), ((), ())),
                              preferred_element_type=jnp.float32)
    out_ref[...] = out * dinv_ref[...]


def _layer(p, y, dinv, b, w):
    return pl.pallas_call(
        _layer_body,
        grid=(GRID,),
        in_specs=[
            pl.BlockSpec((2, BLK, D), lambda i: (0, i, 0)),
            pl.BlockSpec((BLK, D), lambda i: (i, 0)),
            pl.BlockSpec((BLK, 1), lambda i: (i, 0)),
            pl.BlockSpec((1, D), lambda i: (0, 0)),
            pl.BlockSpec((D, D), lambda i: (0, 0)),
        ],
        out_specs=pl.BlockSpec((BLK, D), lambda i: (i, 0)),
        out_shape=jax.ShapeDtypeStruct((NP, D), jnp.float32),
    )(p, y, dinv, b, w)


def _readout_body(p_ref, y_ref, dinv_ref, b_ref, batch_ref, w1_ref, b1_ref,
                  w2_ref, b2_ref, out_ref, gmax, gsum, cnt):
    i = pl.program_id(0)

    @pl.when(i == 0)
    def _init():
        gmax[...] = jnp.full((G, D), -jnp.inf, jnp.float32)
        gsum[...] = jnp.zeros((G, D), jnp.float32)
        cnt[...] = jnp.zeros((G, 1), jnp.float32)

    agg = p_ref[0] + p_ref[1] + y_ref[...]
    h = jnp.maximum(agg * dinv_ref[...] + b_ref[...], 0.0)

    bidx = batch_ref[...]
    giota = lax.broadcasted_iota(jnp.int32, (BLK, G), 1)
    onehot = (bidx == giota).astype(jnp.float32)
    gsum[...] += jax.lax.dot_general(onehot, h, (((0,), (0,)), ((), ())),
                                     preferred_element_type=jnp.float32)
    cnt[...] += jax.lax.dot_general(
        onehot, jnp.ones((BLK, 1), jnp.float32), (((0,), (0,)), ((), ())),
        preferred_element_type=jnp.float32)

    def upd(g, _):
        row = jnp.max(jnp.where(bidx == g, h, -jnp.inf), axis=0, keepdims=True)
        gmax[pl.ds(g, 1), :] = jnp.maximum(gmax[pl.ds(g, 1), :], row)
        return 0
    lax.fori_loop(0, G, upd, 0)

    @pl.when(i == GRID - 1)
    def _fin():
        gmean = gsum[...] / jnp.maximum(cnt[...], 1.0)
        hcat = jnp.concatenate([gmax[...], gmean], axis=1)
        h1 = jax.lax.dot_general(hcat, w1_ref[...], (((1,), (0,)), ((), ())),
                                 preferred_element_type=jnp.float32)
        h1 = jnp.maximum(h1 + b1_ref[...], 0.0)
        out = jax.lax.dot_general(h1, w2_ref[...], (((1,), (0,)), ((), ())),
                                  preferred_element_type=jnp.float32)
        out_ref[...] = out + b2_ref[...]


def _readout(p, y, dinv, b, batch_col, w1, b1, w2, b2):
    return pl.pallas_call(
        _readout_body,
        grid=(GRID,),
        in_specs=[
            pl.BlockSpec((2, BLK, D), lambda i: (0, i, 0)),
            pl.BlockSpec((BLK, D), lambda i: (i, 0)),
            pl.BlockSpec((BLK, 1), lambda i: (i, 0)),
            pl.BlockSpec((1, D), lambda i: (0, 0)),
            pl.BlockSpec((BLK, 1), lambda i: (i, 0)),
            pl.BlockSpec((2 * D, D), lambda i: (0, 0)),
            pl.BlockSpec((1, D), lambda i: (0, 0)),
            pl.BlockSpec((D, 1), lambda i: (0, 0)),
            pl.BlockSpec((1, 1), lambda i: (0, 0)),
        ],
        out_specs=pl.BlockSpec((G, 1), lambda i: (0, 0)),
        out_shape=jax.ShapeDtypeStruct((G, 1), jnp.float32),
        scratch_shapes=[
            pltpu.VMEM((G, D), jnp.float32),
            pltpu.VMEM((G, D), jnp.float32),
            pltpu.VMEM((G, 1), jnp.float32),
        ],
    )(p, y, dinv, b, batch_col, w1, b1, w2, b2)


_sc_mesh = plsc.VectorSubcoreMesh(core_axis_name="c", subcore_axis_name="s")


def _sc_deg(dsts, ones, zeros):
    @functools.partial(
        pl.kernel, mesh=_sc_mesh,
        out_type=jax.ShapeDtypeStruct((2, NP, DW), jnp.float32),
        scratch_types=[
            pltpu.VMEM((NCH, K), jnp.int32),
            pltpu.VMEM((K, DW), jnp.float32),
            pltpu.VMEM_SHARED((NP, DW), jnp.float32),
            pltpu.SemaphoreType.DMA,
        ],
    )
    def deg_kernel(dst_hbm, ones_hbm, z_hbm, out_hbm, dst_v, ones_v, acc, sem):
        c = lax.axis_index("c")
        s = lax.axis_index("s")
        w = s * 2 + c
        pltpu.sync_copy(dst_hbm.at[w], dst_v)
        pltpu.sync_copy(ones_hbm, ones_v)
        pltpu.sync_copy(z_hbm.at[pl.ds(s * RPS, RPS)], acc.at[pl.ds(s * RPS, RPS)])
        plsc.subcore_barrier()

        def body(gi, carry):
            base = gi * DEGG
            descs = [pltpu.async_copy(ones_v, acc.at[dst_v.at[base + b]],
                                      sem, add=True) for b in range(DEGG)]
            for d in descs:
                d.wait()
            return carry
        lax.fori_loop(0, NCH // DEGG, body, 0)
        plsc.subcore_barrier()
        pltpu.sync_copy(acc.at[pl.ds(s * RPS, RPS)],
                        out_hbm.at[c, pl.ds(s * RPS, RPS)])

    return deg_kernel(dsts, ones, zeros)


def _sc_agg(y, srcs, dsts, zeros):
    @functools.partial(
        pl.kernel, mesh=_sc_mesh,
        out_type=jax.ShapeDtypeStruct((2, NP, D), jnp.float32),
        scratch_types=(
            [pltpu.VMEM((NCH, K), jnp.int32)]
            + [pltpu.VMEM((K,), jnp.int32) for _ in range(2 * NBUF)]
            + [pltpu.VMEM((K, D), jnp.float32) for _ in range(NBUF)]
            + [pltpu.VMEM_SHARED((NP, D), jnp.float32)]
            + [pltpu.SemaphoreType.DMA for _ in range(3)]
        ),
    )
    def agg_kernel(y_hbm, src_hbm, dst_hbm, z_hbm, out_hbm, *rest):
        dst_v = rest[0]
        sidxA = rest[1:1 + NBUF]
        sidxB = rest[1 + NBUF:1 + 2 * NBUF]
        bufs = rest[1 + 2 * NBUF:1 + 3 * NBUF]
        acc = rest[1 + 3 * NBUF]
        gsem = rest[2 + 3 * NBUF]
        ssem = rest[3 + 3 * NBUF]
        isem = rest[4 + 3 * NBUF]
        c = lax.axis_index("c")
        s = lax.axis_index("s")
        w = s * 2 + c
        pltpu.sync_copy(dst_hbm.at[w], dst_v)
        pltpu.sync_copy(z_hbm.at[pl.ds(s * RPS, RPS)], acc.at[pl.ds(s * RPS, RPS)])
        plsc.subcore_barrier()

        for b in range(NBUF):
            pltpu.sync_copy(src_hbm.at[w, b], sidxA[b])
            pltpu.async_copy(y_hbm.at[sidxA[b]], bufs[b], gsem)
            pltpu.async_copy(src_hbm.at[w, NBUF + b], sidxB[b], isem)

        def half(base, cur, nxt):
            descs = []
            for b in range(NBUF):
                pltpu.make_async_copy(y_hbm.at[cur[b]], bufs[b], gsem).wait()
                descs.append(pltpu.async_copy(bufs[b], acc.at[dst_v.at[base + b]],
                                              ssem, add=True))
            for b in range(NBUF):
                descs[b].wait()
                gj = base + NBUF + b
                ij = base + 2 * NBUF + b

                @pl.when(gj < NCH)
                def _fire():
                    pltpu.make_async_copy(src_hbm.at[w, gj], nxt[b], isem).wait()
                    pltpu.async_copy(y_hbm.at[nxt[b]], bufs[b], gsem)

                @pl.when(ij < NCH)
                def _prefetch_idx():
                    pltpu.async_copy(src_hbm.at[w, ij], cur[b], isem)

        def body(t, carry):
            half(2 * NBUF * t, sidxA, sidxB)
            half(2 * NBUF * t + NBUF, sidxB, sidxA)
            return carry
        lax.fori_loop(0, NCH // (2 * NBUF), body, 0)
        plsc.subcore_barrier()
        pltpu.sync_copy(acc.at[pl.ds(s * RPS, RPS)],
                        out_hbm.at[c, pl.ds(s * RPS, RPS)])

    return agg_kernel(y, srcs, dsts, zeros)


def kernel(x, edge_index, batch_index, W0, b0, W1, b1, W2, b2, W3, b3, W4, b4,
           out1_W, out1_b, out2_W, out2_b):
    srcs = edge_index[0].astype(jnp.int32).reshape(NW, NCH, K)
    dsts = edge_index[1].astype(jnp.int32).reshape(NW, NCH, K)
    xp = jnp.pad(x, ((0, NP - N), (0, 0)))
    batch_p = jnp.pad(batch_index.astype(jnp.int32), (0, NP - N),
                      constant_values=G)
    zeros = jnp.zeros((NP, D), jnp.float32)
    ones = jnp.ones((K, DW), jnp.float32)
    zeros_dw = jnp.zeros((NP, DW), jnp.float32)

    deg_parts = _sc_deg(dsts, ones, zeros_dw)
    dinv, y = _prep(deg_parts, xp, W0)

    bs = [b0, b1, b2, b3]
    ws = [W1, W2, W3, W4]
    for k in range(4):
        p = _sc_agg(y, srcs, dsts, zeros)
        y = _layer(p, y, dinv, bs[k].reshape(1, D), ws[k])

    p = _sc_agg(y, srcs, dsts, zeros)
    out = _readout(p, y, dinv, b4.reshape(1, D), batch_p.reshape(NP, 1),
                   out1_W, out1_b.reshape(1, D), out2_W, out2_b.reshape(1, 1))
    return out
